# baseline (speedup 1.0000x reference)
import functools
import math

import jax
import jax.numpy as jnp
from jax import lax
from jax.experimental import pallas as pl
from jax.experimental.pallas import tpu as pltpu
from jax.experimental.pallas import tpu_sc as plsc

NPRED = 1_000_000
BATCH = 16384
DUR_N = 11
TOTAL = float(DUR_N - 1)

_NC, _NS = 2, 16
_NW = _NC * _NS
_BPW = BATCH // _NW

_W = 2 * NPRED
_PW = 62496
_CH = 10416
_NCHW = 6
_NV = _CH // 16
_TAIL_OFF = _PW * _NW
_TAIL = _W - _TAIL_OFF

_EPS = float(jnp.finfo(jnp.float32).eps)
_LOGC = [
    math.lgamma(DUR_N) - math.lgamma(j + 1.0) - math.lgamma(TOTAL - j + 1.0)
    for j in range(DUR_N)
]


def _sc_min_gather(pred1d, idx1, idx2):
    mesh = plsc.VectorSubcoreMesh(core_axis_name="c", subcore_axis_name="s")

    @functools.partial(
        pl.kernel,
        mesh=mesh,
        out_type=[
            jax.ShapeDtypeStruct((_NW * 16,), jnp.float32),
            jax.ShapeDtypeStruct((4, 128, 128), jnp.float32),
        ],
        scratch_types=[
            pltpu.VMEM((_BPW,), jnp.int32),
            pltpu.VMEM((4, 4, 128), jnp.int32),
            pltpu.VMEM((16, 128), jnp.float32),
            pltpu.VMEM((_CH,), jnp.float32),
            pltpu.VMEM((16,), jnp.float32),
            pltpu.SemaphoreType.DMA,
        ],
    )
    def ka(tab, i1, i2, pm_out, g_out, raw_v, sidx_v, rows_v, buf_v, mv_v,
           gsem):
        wid = lax.axis_index("s") * _NC + lax.axis_index("c")
        for t, src in enumerate((i1, i2)):
            pltpu.sync_copy(src.at[pl.ds(wid * _BPW, _BPW)], raw_v)
            for parity in range(2):
                r = 2 * t + parity
                for i in range(_BPW // 16):
                    v = raw_v[pl.ds(i * 16, 16)]
                    sidx_v[r, i // 8, pl.ds((i % 8) * 16, 16)] = v * 2 + parity
        gd = [
            pltpu.async_copy(tab.at[sidx_v.at[r, j]], rows_v.at[4 * r + j],
                             gsem)
            for r in range(4) for j in range(4)
        ]
        evenmask = lax.broadcasted_iota(jnp.int32, (16,), 0) % 2 == 0
        inf16 = jnp.full((16,), jnp.inf, jnp.float32)
        wbase = wid * _PW
        acc = inf16
        for c in range(_NCHW):
            pltpu.sync_copy(tab.at[pl.ds(wbase + c * _CH, _CH)], buf_v)

            def body(i, a):
                v = buf_v[pl.ds(i * 16, 16)]
                sq = v * v
                return jnp.minimum(a, jnp.where(evenmask, sq, inf16))

            acc = lax.fori_loop(0, _NV, body, acc)
        pltpu.sync_copy(tab.at[pl.ds(_TAIL_OFF, _TAIL)],
                        buf_v.at[pl.ds(0, _TAIL)])

        def tbody(i, a):
            v = buf_v[pl.ds(i * 16, 16)]
            sq = v * v
            return jnp.minimum(a, jnp.where(evenmask, sq, inf16))

        acc = lax.fori_loop(0, _TAIL // 16, tbody, acc)
        mv_v[...] = acc
        pltpu.sync_copy(mv_v, pm_out.at[pl.ds(wid * 16, 16)])
        for d in gd:
            d.wait()
        for r in range(4):
            pltpu.sync_copy(rows_v.at[pl.ds(4 * r, 4), :],
                            g_out.at[r, pl.ds(wid * 4, 4), :])

    return ka(pred1d, idx1, idx2)


def _sc_anchor(pred1d, partmin):
    mesh = plsc.VectorSubcoreMesh(core_axis_name="c", subcore_axis_name="s")

    @functools.partial(
        pl.kernel,
        mesh=mesh,
        out_type=jax.ShapeDtypeStruct((_W,), jnp.float32),
        scratch_types=[
            pltpu.VMEM((_NW * 16,), jnp.float32),
            pltpu.VMEM((_CH,), jnp.float32),
            pltpu.VMEM((_CH,), jnp.float32),
        ],
    )
    def kb(tab, pm, anch_out, pmv, ibuf, obuf):
        wid = lax.axis_index("s") * _NC + lax.axis_index("c")
        pltpu.sync_copy(pm, pmv)
        macc = jnp.full((16,), jnp.inf, jnp.float32)
        for i in range(_NW):
            macc = jnp.minimum(macc, pmv[pl.ds(i * 16, 16)])
        lane = lax.broadcasted_iota(jnp.int32, (16,), 0)
        m16 = macc
        dnums = lax.GatherDimensionNumbers(
            offset_dims=(), collapsed_slice_dims=(0,), start_index_map=(0,))
        for s in (1, 2, 4, 8):
            rot = lax.gather(
                m16, ((lane + s) & 15)[:, None], dnums, slice_sizes=(1,),
                mode=lax.GatherScatterMode.PROMISE_IN_BOUNDS)
            m16 = jnp.minimum(m16, rot)
        evenmask = lax.broadcasted_iota(jnp.int32, (16,), 0) % 2 == 0
        wbase = wid * _PW
        for c in range(_NCHW):
            pltpu.sync_copy(tab.at[pl.ds(wbase + c * _CH, _CH)], ibuf)

            def body(i, _):
                v = ibuf[pl.ds(i * 16, 16)]
                sq = v * v
                obuf[pl.ds(i * 16, 16)] = jnp.where(evenmask, sq - m16, sq)
                return 0

            lax.fori_loop(0, _NV, body, 0)
            pltpu.sync_copy(obuf, anch_out.at[pl.ds(wbase + c * _CH, _CH)])

        @pl.when(wid == _NW - 1)
        def _():
            pltpu.sync_copy(tab.at[pl.ds(_TAIL_OFF, _TAIL)],
                            ibuf.at[pl.ds(0, _TAIL)])

            def tbody(i, _):
                v = ibuf[pl.ds(i * 16, 16)]
                sq = v * v
                obuf[pl.ds(i * 16, 16)] = jnp.where(evenmask, sq - m16, sq)
                return 0

            lax.fori_loop(0, _TAIL // 16, tbody, 0)
            pltpu.sync_copy(obuf.at[pl.ds(0, _TAIL)],
                            anch_out.at[pl.ds(_TAIL_OFF, _TAIL)])

    return kb(pred1d, partmin)


def _small_body(pm_ref, k_ref, g_ref,
                b1_ref, d1_ref, b2_ref, d2_ref, p1_ref, p2_ref):
    minv = jnp.min(pm_ref[...])
    kk = k_ref[0, 0]
    for t, (b_ref, d_ref, p_ref) in enumerate(
            ((b1_ref, d1_ref, p1_ref), (b2_ref, d2_ref, p2_ref))):
        a = g_ref[2 * t]
        d = g_ref[2 * t + 1]
        dur = d * d
        b_ref[...] = a * a - minv
        d_ref[...] = dur
        x = kk * jnp.log(dur)
        p = jax.nn.sigmoid(x)
        p = jnp.clip(p, _EPS, 1.0 - _EPS)
        logits = jnp.log(p) - jnp.log1p(-p)
        neg_max = jnp.minimum(logits, 0.0)
        base = TOTAL * neg_max - TOTAL * jnp.log(
            jnp.exp(neg_max) + jnp.exp(-logits + neg_max))
        for j in range(DUR_N):
            p_ref[:, :, j] = _LOGC[j] + float(j) * logits + base


def kernel(idx1, idx2, pred_tensor, k):
    pred1d = pred_tensor.reshape(_W)

    partmin, g = _sc_min_gather(pred1d, idx1, idx2)
    anch1d = _sc_anchor(pred1d, partmin)

    k2 = k.reshape(1, 1)
    grid_s = 16
    sub = 128 // grid_s
    b1, d1, b2, d2, p1, p2 = pl.pallas_call(
        _small_body,
        grid=(grid_s,),
        in_specs=[
            pl.BlockSpec((_NW * 16,), lambda i: (0,)),
            pl.BlockSpec(memory_space=pltpu.SMEM),
            pl.BlockSpec((4, sub, 128), lambda i: (0, i, 0)),
        ],
        out_specs=[
            pl.BlockSpec((sub, 128), lambda i: (i, 0)),
            pl.BlockSpec((sub, 128), lambda i: (i, 0)),
            pl.BlockSpec((sub, 128), lambda i: (i, 0)),
            pl.BlockSpec((sub, 128), lambda i: (i, 0)),
            pl.BlockSpec((sub, 128, DUR_N), lambda i: (i, 0, 0)),
            pl.BlockSpec((sub, 128, DUR_N), lambda i: (i, 0, 0)),
        ],
        out_shape=[
            jax.ShapeDtypeStruct((128, 128), jnp.float32),
            jax.ShapeDtypeStruct((128, 128), jnp.float32),
            jax.ShapeDtypeStruct((128, 128), jnp.float32),
            jax.ShapeDtypeStruct((128, 128), jnp.float32),
            jax.ShapeDtypeStruct((128, 128, DUR_N), jnp.float32),
            jax.ShapeDtypeStruct((128, 128, DUR_N), jnp.float32),
        ],
    )(partmin, k2, g)

    return (
        b1.reshape(BATCH),
        d1.reshape(BATCH),
        b2.reshape(BATCH),
        d2.reshape(BATCH),
        p1.reshape(BATCH, DUR_N),
        p2.reshape(BATCH, DUR_N),
        anch1d.reshape(NPRED, 2),
    )


# speedup vs baseline: 4.9830x; 4.9830x over previous
import functools
import math

import jax
import jax.numpy as jnp
from jax import lax
from jax.experimental import pallas as pl
from jax.experimental.pallas import tpu as pltpu
from jax.experimental.pallas import tpu_sc as plsc

NPRED = 1_000_000
BATCH = 16384
DUR_N = 11
TOTAL = float(DUR_N - 1)

_NSB = 7812
_MAINROWS = _NSB * 128
_MAINW = _NSB * 256
_VROWS = 2 * _NSB
_TAILN = NPRED - _MAINROWS

_RBLK = 1736
_GRID = _VROWS // _RBLK

_NC, _NS = 2, 16
_NW = _NC * _NS
_BPW = BATCH // _NW

_EPS = float(jnp.finfo(jnp.float32).eps)
_LOGC = [
    math.lgamma(DUR_N) - math.lgamma(j + 1.0) - math.lgamma(TOTAL - j + 1.0)
    for j in range(DUR_N)
]


def _min_body(x_ref, t_ref, o_ref, acc_ref):
    i = pl.program_id(0)
    x = x_ref[...]
    sq = x * x
    row = lax.broadcasted_iota(jnp.int32, x.shape, 0)
    m = jnp.min(jnp.where(row % 2 == 0, sq, jnp.inf))

    @pl.when(i == 0)
    def _():
        t = t_ref[...]
        tsq = t * t
        lanep = lax.broadcasted_iota(jnp.int32, t.shape, 1)
        acc_ref[0, 0] = jnp.min(jnp.where(lanep % 2 == 0, tsq, jnp.inf))

    acc_ref[0, 0] = jnp.minimum(acc_ref[0, 0], m)

    @pl.when(i == _GRID - 1)
    def _():
        o_ref[0, 0] = acc_ref[0, 0]


def _anch_body(m_ref, x_ref, o_ref):
    x = x_ref[...]
    sq = x * x
    row = lax.broadcasted_iota(jnp.int32, x.shape, 0)
    o_ref[...] = jnp.where(row % 2 == 0, sq - m_ref[0, 0], sq)


def _tail_body(m_ref, t_ref, o_ref):
    t = t_ref[...]
    tsq = t * t
    lanep = lax.broadcasted_iota(jnp.int32, t.shape, 1)
    o_ref[...] = jnp.where(lanep % 2 == 0, tsq - m_ref[0, 0], tsq)


def _gather_sc(view1d, tail128, idx1, idx2):
    mesh = plsc.VectorSubcoreMesh(core_axis_name="c", subcore_axis_name="s")

    @functools.partial(
        pl.kernel,
        mesh=mesh,
        out_type=jax.ShapeDtypeStruct((4, 128, 128), jnp.float32),
        scratch_types=[
            pltpu.VMEM((2, _BPW), jnp.int32),
            pltpu.VMEM((4, 4, 128), jnp.int32),
            pltpu.VMEM((4, 4, 128), jnp.int32),
            pltpu.VMEM((16, 128), jnp.float32),
            pltpu.VMEM((16, 128), jnp.float32),
            pltpu.SemaphoreType.DMA,
        ],
    )
    def kg(tab, tl, i1, i2, g_out, raw_v, adr_v, tadr_v, rows_v, trows_v,
           sem):
        wid = lax.axis_index("s") * _NC + lax.axis_index("c")
        for t, src in enumerate((i1, i2)):
            pltpu.sync_copy(src.at[pl.ds(wid * _BPW, _BPW)], raw_v.at[t])
            for i in range(_BPW // 16):
                v = raw_v[t, pl.ds(i * 16, 16)]
                a0 = (v >> 7) * 256 + (v & 127)
                a0 = jnp.minimum(a0, _MAINW - 1)
                t0 = jnp.minimum(jnp.maximum((v - _MAINROWS) * 2, 0), 126)
                rr, cc = i // 8, (i % 8) * 16
                for parity in range(2):
                    r = 2 * t + parity
                    adr_v[r, rr, pl.ds(cc, 16)] = a0 + parity * 128
                    tadr_v[r, rr, pl.ds(cc, 16)] = t0 + parity
        gd = [
            pltpu.async_copy(tab.at[adr_v.at[r, j]], rows_v.at[4 * r + j],
                             sem)
            for r in range(4) for j in range(4)
        ] + [
            pltpu.async_copy(tl.at[tadr_v.at[r, j]], trows_v.at[4 * r + j],
                             sem)
            for r in range(4) for j in range(4)
        ]
        for d in gd:
            d.wait()
        for t in range(2):
            for i in range(_BPW // 16):
                v = raw_v[t, pl.ds(i * 16, 16)]
                tmask = v >= _MAINROWS
                for parity in range(2):
                    rr, cc = 4 * (2 * t + parity) + i // 8, (i % 8) * 16
                    mv = rows_v[rr, pl.ds(cc, 16)]
                    tv = trows_v[rr, pl.ds(cc, 16)]
                    rows_v[rr, pl.ds(cc, 16)] = jnp.where(tmask, tv, mv)
        for r in range(4):
            pltpu.sync_copy(rows_v.at[pl.ds(4 * r, 4), :],
                            g_out.at[r, pl.ds(wid * 4, 4), :])

    return kg(view1d, tail128, idx1, idx2)


def _small_body(m_ref, k_ref, g_ref,
                b1_ref, d1_ref, b2_ref, d2_ref, q1_ref, q2_ref):
    minv = m_ref[0, 0]
    kk = k_ref[0, 0]
    for t, (b_ref, d_ref, q_ref) in enumerate(
            ((b1_ref, d1_ref, q1_ref), (b2_ref, d2_ref, q2_ref))):
        a = g_ref[2 * t]
        d = g_ref[2 * t + 1]
        dur = d * d
        b_ref[...] = a * a - minv
        d_ref[...] = dur
        x = kk * jnp.log(dur)
        p = jax.nn.sigmoid(x)
        p = jnp.clip(p, _EPS, 1.0 - _EPS)
        logits = jnp.log(p) - jnp.log1p(-p)
        neg_max = jnp.minimum(logits, 0.0)
        base = TOTAL * neg_max - TOTAL * jnp.log(
            jnp.exp(neg_max) + jnp.exp(-logits + neg_max))
        for j in range(DUR_N):
            q_ref[j] = _LOGC[j] + float(j) * logits + base
        for j in range(DUR_N, 16):
            q_ref[j] = jnp.zeros_like(base)


def kernel(idx1, idx2, pred_tensor, k):
    view2d = (pred_tensor[:_MAINROWS]
              .reshape(_NSB, 128, 2)
              .transpose(0, 2, 1)
              .reshape(_VROWS, 128))
    view1d = view2d.reshape(_MAINW)
    tail128 = pred_tensor[_MAINROWS:].reshape(1, 128)

    g = _gather_sc(view1d, tail128.reshape(128), idx1, idx2)

    minv = pl.pallas_call(
        _min_body,
        grid=(_GRID,),
        in_specs=[
            pl.BlockSpec((_RBLK, 128), lambda i: (i, 0)),
            pl.BlockSpec((1, 128), lambda i: (0, 0)),
        ],
        out_specs=pl.BlockSpec(memory_space=pltpu.SMEM),
        out_shape=jax.ShapeDtypeStruct((1, 1), jnp.float32),
        scratch_shapes=[pltpu.SMEM((1, 1), jnp.float32)],
    )(view2d, tail128)

    def _anch_full(m_ref, x_ref, t_ref, o_ref, to_ref):
        _anch_body(m_ref, x_ref, o_ref)
        _tail_body(m_ref, t_ref, to_ref)

    anch_v, tail_out = pl.pallas_call(
        _anch_full,
        grid=(_GRID,),
        in_specs=[
            pl.BlockSpec(memory_space=pltpu.SMEM),
            pl.BlockSpec((_RBLK, 128), lambda i: (i, 0)),
            pl.BlockSpec((1, 128), lambda i: (0, 0)),
        ],
        out_specs=[
            pl.BlockSpec((_RBLK, 128), lambda i: (i, 0)),
            pl.BlockSpec((1, 128), lambda i: (0, 0)),
        ],
        out_shape=[
            jax.ShapeDtypeStruct((_VROWS, 128), jnp.float32),
            jax.ShapeDtypeStruct((1, 128), jnp.float32),
        ],
    )(minv, view2d, tail128)

    main_view = (anch_v.reshape(_NSB, 2, 128)
                 .transpose(0, 2, 1)
                 .reshape(_MAINROWS, 2))
    anchored = jnp.concatenate(
        [main_view, tail_out.reshape(_TAILN, 2)], axis=0)

    k2 = k.reshape(1, 1)
    b1, d1, b2, d2, q1, q2 = pl.pallas_call(
        _small_body,
        grid=(16,),
        in_specs=[
            pl.BlockSpec(memory_space=pltpu.SMEM),
            pl.BlockSpec(memory_space=pltpu.SMEM),
            pl.BlockSpec((4, 8, 128), lambda i: (0, i, 0)),
        ],
        out_specs=[
            pl.BlockSpec((8, 128), lambda i: (i, 0)),
            pl.BlockSpec((8, 128), lambda i: (i, 0)),
            pl.BlockSpec((8, 128), lambda i: (i, 0)),
            pl.BlockSpec((8, 128), lambda i: (i, 0)),
            pl.BlockSpec((16, 8, 128), lambda i: (0, i, 0)),
            pl.BlockSpec((16, 8, 128), lambda i: (0, i, 0)),
        ],
        out_shape=[
            jax.ShapeDtypeStruct((128, 128), jnp.float32),
            jax.ShapeDtypeStruct((128, 128), jnp.float32),
            jax.ShapeDtypeStruct((128, 128), jnp.float32),
            jax.ShapeDtypeStruct((128, 128), jnp.float32),
            jax.ShapeDtypeStruct((16, 128, 128), jnp.float32),
            jax.ShapeDtypeStruct((16, 128, 128), jnp.float32),
        ],
    )(minv, k2, g)

    p1 = q1.reshape(16, BATCH).T[:, :DUR_N]
    p2 = q2.reshape(16, BATCH).T[:, :DUR_N]

    return (
        b1.reshape(BATCH),
        d1.reshape(BATCH),
        b2.reshape(BATCH),
        d2.reshape(BATCH),
        p1,
        p2,
        anchored,
    )


# speedup vs baseline: 18.2262x; 3.6577x over previous
import functools
import math

import jax
import jax.numpy as jnp
from jax import lax
from jax.experimental import pallas as pl
from jax.experimental.pallas import tpu as pltpu
from jax.experimental.pallas import tpu_sc as plsc

NPRED = 1_000_000
BATCH = 16384
DUR_N = 11
TOTAL = float(DUR_N - 1)

_NSB = 7812
_MAINROWS = _NSB * 128
_MAINW = _NSB * 256
_VROWS = 2 * _NSB
_TAILN = NPRED - _MAINROWS

_RBLK = 1736
_GRID = _VROWS // _RBLK

_NC, _NS = 2, 16
_NW = _NC * _NS
_BPW = BATCH // _NW

_EPS = float(jnp.finfo(jnp.float32).eps)
_LOGC = [
    math.lgamma(DUR_N) - math.lgamma(j + 1.0) - math.lgamma(TOTAL - j + 1.0)
    for j in range(DUR_N)
]


def _min_body(x_ref, t_ref, o_ref, acc_ref):
    i = pl.program_id(0)
    x = x_ref[...]
    sq = x * x
    row = lax.broadcasted_iota(jnp.int32, x.shape, 0)
    m = jnp.min(jnp.where(row % 2 == 0, sq, jnp.inf))

    @pl.when(i == 0)
    def _():
        t = t_ref[...]
        tsq = t * t
        lanep = lax.broadcasted_iota(jnp.int32, t.shape, 1)
        acc_ref[0, 0] = jnp.min(jnp.where(lanep % 2 == 0, tsq, jnp.inf))

    acc_ref[0, 0] = jnp.minimum(acc_ref[0, 0], m)

    @pl.when(i == _GRID - 1)
    def _():
        o_ref[0, 0] = acc_ref[0, 0]


def _anch_body(m_ref, x_ref, o_ref):
    x = x_ref[...]
    sq = x * x
    row = lax.broadcasted_iota(jnp.int32, x.shape, 0)
    o_ref[...] = jnp.where(row % 2 == 0, sq - m_ref[0, 0], sq)


def _tail_body(m_ref, t_ref, o_ref):
    t = t_ref[...]
    tsq = t * t
    lanep = lax.broadcasted_iota(jnp.int32, t.shape, 1)
    o_ref[...] = jnp.where(lanep % 2 == 0, tsq - m_ref[0, 0], tsq)


def _gather_sc(view1d, tail128, idx1, idx2):
    mesh = plsc.VectorSubcoreMesh(core_axis_name="c", subcore_axis_name="s")

    @functools.partial(
        pl.kernel,
        mesh=mesh,
        out_type=jax.ShapeDtypeStruct((4, 128, 128), jnp.float32),
        scratch_types=[
            pltpu.VMEM((2, _BPW), jnp.int32),
            pltpu.VMEM((4, 4, 128), jnp.int32),
            pltpu.VMEM((16, 128), jnp.float32),
            pltpu.VMEM((128,), jnp.float32),
            pltpu.SemaphoreType.DMA,
        ],
    )
    def kg(tab, tl, i1, i2, g_out, raw_v, adr_v, rows_v, tail_v, sem):
        wid = lax.axis_index("s") * _NC + lax.axis_index("c")
        pltpu.sync_copy(tl, tail_v)
        for t, src in enumerate((i1, i2)):
            pltpu.sync_copy(src.at[pl.ds(wid * _BPW, _BPW)], raw_v.at[t])
            for i in range(_BPW // 16):
                v = raw_v[t, pl.ds(i * 16, 16)]
                a0 = (v >> 7) * 256 + (v & 127)
                a0 = jnp.minimum(a0, _MAINW - 1)
                rr, cc = i // 8, (i % 8) * 16
                for parity in range(2):
                    r = 2 * t + parity
                    adr_v[r, rr, pl.ds(cc, 16)] = a0 + parity * 128
        gd = [
            pltpu.async_copy(tab.at[adr_v.at[r, j]], rows_v.at[4 * r + j],
                             sem)
            for r in range(4) for j in range(4)
        ]
        for d in gd:
            d.wait()
        dnums = lax.GatherDimensionNumbers(
            offset_dims=(), collapsed_slice_dims=(0,), start_index_map=(0,))
        tvs = [tail_v[pl.ds(j * 16, 16)] for j in range(8)]
        for t in range(2):
            for i in range(_BPW // 16):
                v = raw_v[t, pl.ds(i * 16, 16)]
                tmask = v >= _MAINROWS
                t0 = jnp.minimum(jnp.maximum((v - _MAINROWS) * 2, 0), 126)
                for parity in range(2):
                    toff = t0 + parity
                    grp = toff >> 4
                    ln = toff & 15
                    val = jnp.zeros((16,), jnp.float32)
                    for j in range(8):
                        tv = lax.gather(
                            tvs[j], ln[:, None], dnums, slice_sizes=(1,),
                            mode=lax.GatherScatterMode.PROMISE_IN_BOUNDS)
                        val = jnp.where(grp == j, tv, val)
                    rr = 4 * (2 * t + parity) + i // 8
                    cc = (i % 8) * 16
                    mv = rows_v[rr, pl.ds(cc, 16)]
                    rows_v[rr, pl.ds(cc, 16)] = jnp.where(tmask, val, mv)

        for r in range(4):
            pltpu.sync_copy(rows_v.at[pl.ds(4 * r, 4), :],
                            g_out.at[r, pl.ds(wid * 4, 4), :])

    return kg(view1d, tail128, idx1, idx2)


def _small_body(m_ref, k_ref, g_ref,
                b1_ref, d1_ref, b2_ref, d2_ref, q1_ref, q2_ref):
    minv = m_ref[0, 0]
    kk = k_ref[0, 0]
    for t, (b_ref, d_ref, q_ref) in enumerate(
            ((b1_ref, d1_ref, q1_ref), (b2_ref, d2_ref, q2_ref))):
        a = g_ref[2 * t]
        d = g_ref[2 * t + 1]
        dur = d * d
        b_ref[...] = a * a - minv
        d_ref[...] = dur
        x = kk * jnp.log(dur)
        p = jax.nn.sigmoid(x)
        p = jnp.clip(p, _EPS, 1.0 - _EPS)
        logits = jnp.log(p) - jnp.log1p(-p)
        neg_max = jnp.minimum(logits, 0.0)
        base = TOTAL * neg_max - TOTAL * jnp.log(
            jnp.exp(neg_max) + jnp.exp(-logits + neg_max))
        for j in range(DUR_N):
            q_ref[j] = _LOGC[j] + float(j) * logits + base
        for j in range(DUR_N, 16):
            q_ref[j] = jnp.zeros_like(base)


def kernel(idx1, idx2, pred_tensor, k):
    view2d = (pred_tensor[:_MAINROWS]
              .reshape(_NSB, 128, 2)
              .transpose(0, 2, 1)
              .reshape(_VROWS, 128))
    view1d = view2d.reshape(_MAINW)
    tail128 = pred_tensor[_MAINROWS:].reshape(1, 128)

    g = _gather_sc(view1d, tail128.reshape(128), idx1, idx2)

    minv = pl.pallas_call(
        _min_body,
        grid=(_GRID,),
        in_specs=[
            pl.BlockSpec((_RBLK, 128), lambda i: (i, 0)),
            pl.BlockSpec((1, 128), lambda i: (0, 0)),
        ],
        out_specs=pl.BlockSpec(memory_space=pltpu.SMEM),
        out_shape=jax.ShapeDtypeStruct((1, 1), jnp.float32),
        scratch_shapes=[pltpu.SMEM((1, 1), jnp.float32)],
    )(view2d, tail128)

    def _anch_full(m_ref, x_ref, t_ref, o_ref, to_ref):
        _anch_body(m_ref, x_ref, o_ref)
        _tail_body(m_ref, t_ref, to_ref)

    anch_v, tail_out = pl.pallas_call(
        _anch_full,
        grid=(_GRID,),
        in_specs=[
            pl.BlockSpec(memory_space=pltpu.SMEM),
            pl.BlockSpec((_RBLK, 128), lambda i: (i, 0)),
            pl.BlockSpec((1, 128), lambda i: (0, 0)),
        ],
        out_specs=[
            pl.BlockSpec((_RBLK, 128), lambda i: (i, 0)),
            pl.BlockSpec((1, 128), lambda i: (0, 0)),
        ],
        out_shape=[
            jax.ShapeDtypeStruct((_VROWS, 128), jnp.float32),
            jax.ShapeDtypeStruct((1, 128), jnp.float32),
        ],
    )(minv, view2d, tail128)

    main_view = (anch_v.reshape(_NSB, 2, 128)
                 .transpose(0, 2, 1)
                 .reshape(_MAINROWS, 2))
    anchored = jnp.concatenate(
        [main_view, tail_out.reshape(_TAILN, 2)], axis=0)

    k2 = k.reshape(1, 1)
    b1, d1, b2, d2, q1, q2 = pl.pallas_call(
        _small_body,
        grid=(16,),
        in_specs=[
            pl.BlockSpec(memory_space=pltpu.SMEM),
            pl.BlockSpec(memory_space=pltpu.SMEM),
            pl.BlockSpec((4, 8, 128), lambda i: (0, i, 0)),
        ],
        out_specs=[
            pl.BlockSpec((8, 128), lambda i: (i, 0)),
            pl.BlockSpec((8, 128), lambda i: (i, 0)),
            pl.BlockSpec((8, 128), lambda i: (i, 0)),
            pl.BlockSpec((8, 128), lambda i: (i, 0)),
            pl.BlockSpec((16, 8, 128), lambda i: (0, i, 0)),
            pl.BlockSpec((16, 8, 128), lambda i: (0, i, 0)),
        ],
        out_shape=[
            jax.ShapeDtypeStruct((128, 128), jnp.float32),
            jax.ShapeDtypeStruct((128, 128), jnp.float32),
            jax.ShapeDtypeStruct((128, 128), jnp.float32),
            jax.ShapeDtypeStruct((128, 128), jnp.float32),
            jax.ShapeDtypeStruct((16, 128, 128), jnp.float32),
            jax.ShapeDtypeStruct((16, 128, 128), jnp.float32),
        ],
    )(minv, k2, g)

    p1 = q1.reshape(16, BATCH).T[:, :DUR_N]
    p2 = q2.reshape(16, BATCH).T[:, :DUR_N]

    return (
        b1.reshape(BATCH),
        d1.reshape(BATCH),
        b2.reshape(BATCH),
        d2.reshape(BATCH),
        p1,
        p2,
        anchored,
    )


# speedup vs baseline: 31.8432x; 1.7471x over previous
import functools
import math

import jax
import jax.numpy as jnp
from jax import lax
from jax.experimental import pallas as pl
from jax.experimental.pallas import tpu as pltpu
from jax.experimental.pallas import tpu_sc as plsc

NPRED = 1_000_000
BATCH = 16384
DUR_N = 11
TOTAL = float(DUR_N - 1)

_PSTR = 7936
_PROWS = _PSTR * 128
_VR = 2 * _PSTR
_PW = _PSTR * 256
_RBLK = 1984
_GRID = _VR // _RBLK

_NC, _NS = 2, 16
_NW = _NC * _NS
_BPW = BATCH // _NW

_EPS = float(jnp.finfo(jnp.float32).eps)
_LOGC = [
    math.lgamma(DUR_N) - math.lgamma(j + 1.0) - math.lgamma(TOTAL - j + 1.0)
    for j in range(DUR_N)
]


def _min_body(x_ref, o_ref, acc_ref):
    i = pl.program_id(0)
    x = x_ref[...]
    sq = x * x
    row = lax.broadcasted_iota(jnp.int32, x.shape, 0)
    m = jnp.min(jnp.where(row % 2 == 0, sq, jnp.inf))

    @pl.when(i == 0)
    def _():
        acc_ref[0, 0] = m

    acc_ref[0, 0] = jnp.minimum(acc_ref[0, 0], m)

    @pl.when(i == _GRID - 1)
    def _():
        o_ref[0, 0] = acc_ref[0, 0]


def _anch_body(m_ref, x_ref, o_ref):
    x = x_ref[...]
    sq = x * x
    row = lax.broadcasted_iota(jnp.int32, x.shape, 0)
    o_ref[...] = jnp.where(row % 2 == 0, sq - m_ref[0, 0], sq)


def _gather_sc(view1d, idx1, idx2):
    mesh = plsc.VectorSubcoreMesh(core_axis_name="c", subcore_axis_name="s")

    @functools.partial(
        pl.kernel,
        mesh=mesh,
        out_type=jax.ShapeDtypeStruct((4, 128, 128), jnp.float32),
        scratch_types=[
            pltpu.VMEM((_BPW,), jnp.int32),
            pltpu.VMEM((4, 4, 128), jnp.int32),
            pltpu.VMEM((16, 128), jnp.float32),
            pltpu.SemaphoreType.DMA,
        ],
    )
    def kg(tab, i1, i2, g_out, raw_v, adr_v, rows_v, sem):
        wid = lax.axis_index("s") * _NC + lax.axis_index("c")
        for t, src in enumerate((i1, i2)):
            pltpu.sync_copy(src.at[pl.ds(wid * _BPW, _BPW)], raw_v)
            for i in range(_BPW // 16):
                v = raw_v[pl.ds(i * 16, 16)]
                a0 = (v >> 7) * 256 + (v & 127)
                rr, cc = i // 8, (i % 8) * 16
                adr_v[2 * t, rr, pl.ds(cc, 16)] = a0
                adr_v[2 * t + 1, rr, pl.ds(cc, 16)] = a0 + 128
        gd = [
            pltpu.async_copy(tab.at[adr_v.at[r, j]], rows_v.at[4 * r + j],
                             sem)
            for r in range(4) for j in range(4)
        ]
        for d in gd:
            d.wait()
        for r in range(4):
            pltpu.sync_copy(rows_v.at[pl.ds(4 * r, 4), :],
                            g_out.at[r, pl.ds(wid * 4, 4), :])

    return kg(view1d, idx1, idx2)


def _small_body(m_ref, k_ref, g_ref,
                b1_ref, d1_ref, b2_ref, d2_ref, q1_ref, q2_ref):
    minv = m_ref[0, 0]
    kk = k_ref[0, 0]
    for t, (b_ref, d_ref, q_ref) in enumerate(
            ((b1_ref, d1_ref, q1_ref), (b2_ref, d2_ref, q2_ref))):
        a = g_ref[2 * t]
        d = g_ref[2 * t + 1]
        dur = d * d
        b_ref[...] = a * a - minv
        d_ref[...] = dur
        x = kk * jnp.log(dur)
        p = jax.nn.sigmoid(x)
        p = jnp.clip(p, _EPS, 1.0 - _EPS)
        logits = jnp.log(p) - jnp.log1p(-p)
        neg_max = jnp.minimum(logits, 0.0)
        base = TOTAL * neg_max - TOTAL * jnp.log(
            jnp.exp(neg_max) + jnp.exp(-logits + neg_max))
        for j in range(DUR_N):
            q_ref[j] = _LOGC[j] + float(j) * logits + base
        for j in range(DUR_N, 16):
            q_ref[j] = jnp.zeros_like(base)


def kernel(idx1, idx2, pred_tensor, k):
    padded = jnp.pad(pred_tensor, ((0, _PROWS - NPRED), (0, 0)),
                     constant_values=jnp.inf)
    viewp = (padded.reshape(_PSTR, 128, 2)
             .transpose(0, 2, 1)
             .reshape(_VR, 128))
    view1d = viewp.reshape(_PW)

    g = _gather_sc(view1d, idx1, idx2)

    minv = pl.pallas_call(
        _min_body,
        grid=(_GRID,),
        in_specs=[pl.BlockSpec((_RBLK, 128), lambda i: (i, 0))],
        out_specs=pl.BlockSpec(memory_space=pltpu.SMEM),
        out_shape=jax.ShapeDtypeStruct((1, 1), jnp.float32),
        scratch_shapes=[pltpu.SMEM((1, 1), jnp.float32)],
    )(viewp)

    anch_v = pl.pallas_call(
        _anch_body,
        grid=(_GRID,),
        in_specs=[
            pl.BlockSpec(memory_space=pltpu.SMEM),
            pl.BlockSpec((_RBLK, 128), lambda i: (i, 0)),
        ],
        out_specs=pl.BlockSpec((_RBLK, 128), lambda i: (i, 0)),
        out_shape=jax.ShapeDtypeStruct((_VR, 128), jnp.float32),
    )(minv, viewp)

    anchored = (anch_v.reshape(_PSTR, 2, 128)
                .transpose(0, 2, 1)
                .reshape(_PROWS, 2)[:NPRED])

    k2 = k.reshape(1, 1)
    b1, d1, b2, d2, q1, q2 = pl.pallas_call(
        _small_body,
        grid=(16,),
        in_specs=[
            pl.BlockSpec(memory_space=pltpu.SMEM),
            pl.BlockSpec(memory_space=pltpu.SMEM),
            pl.BlockSpec((4, 8, 128), lambda i: (0, i, 0)),
        ],
        out_specs=[
            pl.BlockSpec((8, 128), lambda i: (i, 0)),
            pl.BlockSpec((8, 128), lambda i: (i, 0)),
            pl.BlockSpec((8, 128), lambda i: (i, 0)),
            pl.BlockSpec((8, 128), lambda i: (i, 0)),
            pl.BlockSpec((16, 8, 128), lambda i: (0, i, 0)),
            pl.BlockSpec((16, 8, 128), lambda i: (0, i, 0)),
        ],
        out_shape=[
            jax.ShapeDtypeStruct((128, 128), jnp.float32),
            jax.ShapeDtypeStruct((128, 128), jnp.float32),
            jax.ShapeDtypeStruct((128, 128), jnp.float32),
            jax.ShapeDtypeStruct((128, 128), jnp.float32),
            jax.ShapeDtypeStruct((16, 128, 128), jnp.float32),
            jax.ShapeDtypeStruct((16, 128, 128), jnp.float32),
        ],
    )(minv, k2, g)

    p1 = q1.reshape(16, BATCH).T[:, :DUR_N]
    p2 = q2.reshape(16, BATCH).T[:, :DUR_N]

    return (
        b1.reshape(BATCH),
        d1.reshape(BATCH),
        b2.reshape(BATCH),
        d2.reshape(BATCH),
        p1,
        p2,
        anchored,
    )


# speedup vs baseline: 32.2945x; 1.0142x over previous
import functools
import math

import jax
import jax.numpy as jnp
from jax import lax
from jax.experimental import pallas as pl
from jax.experimental.pallas import tpu as pltpu
from jax.experimental.pallas import tpu_sc as plsc

NPRED = 1_000_000
BATCH = 16384
DUR_N = 11
TOTAL = float(DUR_N - 1)

_PSTR = 7936
_PROWS = _PSTR * 128
_VR = 2 * _PSTR
_PW = _PSTR * 256
_RBLK = 1984
_GRID = _VR // _RBLK

_NC, _NS = 2, 16
_NW = _NC * _NS
_BPW = BATCH // _NW

_EPS = float(jnp.finfo(jnp.float32).eps)
_LOGC = [
    math.lgamma(DUR_N) - math.lgamma(j + 1.0) - math.lgamma(TOTAL - j + 1.0)
    for j in range(DUR_N)
]


def _minanch_body(x_ref, o_ref, acc_ref):
    g = pl.program_id(0)
    x = x_ref[...]
    sq = x * x
    row = lax.broadcasted_iota(jnp.int32, x.shape, 0)

    @pl.when(g < _GRID)
    def _():
        m = jnp.min(jnp.where(row % 2 == 0, sq, jnp.inf))

        @pl.when(g == 0)
        def _():
            acc_ref[0, 0] = m

        acc_ref[0, 0] = jnp.minimum(acc_ref[0, 0], m)

    @pl.when(g >= _GRID)
    def _():
        o_ref[...] = jnp.where(row % 2 == 0, sq - acc_ref[0, 0], sq)


def _minanch_out(x_ref, o_ref, mo_ref, acc_ref):
    _minanch_body(x_ref, o_ref, acc_ref)

    @pl.when(pl.program_id(0) == _GRID - 1)
    def _():
        mo_ref[0, 0] = acc_ref[0, 0]


def _gather_sc(view1d, idx1, idx2):
    mesh = plsc.VectorSubcoreMesh(core_axis_name="c", subcore_axis_name="s")

    @functools.partial(
        pl.kernel,
        mesh=mesh,
        out_type=jax.ShapeDtypeStruct((4, 128, 128), jnp.float32),
        scratch_types=[
            pltpu.VMEM((_BPW,), jnp.int32),
            pltpu.VMEM((4, 4, 128), jnp.int32),
            pltpu.VMEM((16, 128), jnp.float32),
            pltpu.SemaphoreType.DMA,
        ],
    )
    def kg(tab, i1, i2, g_out, raw_v, adr_v, rows_v, sem):
        wid = lax.axis_index("s") * _NC + lax.axis_index("c")
        for t, src in enumerate((i1, i2)):
            pltpu.sync_copy(src.at[pl.ds(wid * _BPW, _BPW)], raw_v)
            for i in range(_BPW // 16):
                v = raw_v[pl.ds(i * 16, 16)]
                a0 = (v >> 7) * 256 + (v & 127)
                rr, cc = i // 8, (i % 8) * 16
                adr_v[2 * t, rr, pl.ds(cc, 16)] = a0
                adr_v[2 * t + 1, rr, pl.ds(cc, 16)] = a0 + 128
        gd = [
            pltpu.async_copy(tab.at[adr_v.at[r, j]], rows_v.at[4 * r + j],
                             sem)
            for r in range(4) for j in range(4)
        ]
        for d in gd:
            d.wait()
        for r in range(4):
            pltpu.sync_copy(rows_v.at[pl.ds(4 * r, 4), :],
                            g_out.at[r, pl.ds(wid * 4, 4), :])

    return kg(view1d, idx1, idx2)


def _small_body(m_ref, k_ref, g_ref,
                b1_ref, d1_ref, b2_ref, d2_ref, q1_ref, q2_ref):
    minv = m_ref[0, 0]
    kk = k_ref[0, 0]
    for t, (b_ref, d_ref, q_ref) in enumerate(
            ((b1_ref, d1_ref, q1_ref), (b2_ref, d2_ref, q2_ref))):
        a = g_ref[2 * t]
        d = g_ref[2 * t + 1]
        dur = d * d
        b_ref[...] = a * a - minv
        d_ref[...] = dur
        x = kk * jnp.log(dur)
        p = jax.nn.sigmoid(x)
        p = jnp.clip(p, _EPS, 1.0 - _EPS)
        logits = jnp.log(p) - jnp.log1p(-p)
        neg_max = jnp.minimum(logits, 0.0)
        base = TOTAL * neg_max - TOTAL * jnp.log(
            jnp.exp(neg_max) + jnp.exp(-logits + neg_max))
        for j in range(DUR_N):
            q_ref[j] = _LOGC[j] + float(j) * logits + base
        for j in range(DUR_N, 16):
            q_ref[j] = jnp.zeros_like(base)


def kernel(idx1, idx2, pred_tensor, k):
    padded = jnp.pad(pred_tensor, ((0, _PROWS - NPRED), (0, 0)),
                     constant_values=jnp.inf)
    viewp = (padded.reshape(_PSTR, 128, 2)
             .transpose(0, 2, 1)
             .reshape(_VR, 128))
    view1d = viewp.reshape(_PW)

    g = _gather_sc(view1d, idx1, idx2)

    anch_v, minv = pl.pallas_call(
        _minanch_out,
        grid=(2 * _GRID,),
        in_specs=[pl.BlockSpec((_RBLK, 128), lambda g: (g % _GRID, 0))],
        out_specs=[
            pl.BlockSpec((_RBLK, 128),
                         lambda g: ((g >= _GRID) * (g - _GRID), 0)),
            pl.BlockSpec(memory_space=pltpu.SMEM),
        ],
        out_shape=[
            jax.ShapeDtypeStruct((_VR, 128), jnp.float32),
            jax.ShapeDtypeStruct((1, 1), jnp.float32),
        ],
        scratch_shapes=[pltpu.SMEM((1, 1), jnp.float32)],
    )(viewp)

    anchored = (anch_v.reshape(_PSTR, 2, 128)
                .transpose(0, 2, 1)
                .reshape(_PROWS, 2)[:NPRED])

    k2 = k.reshape(1, 1)
    b1, d1, b2, d2, q1, q2 = pl.pallas_call(
        _small_body,
        grid=(16,),
        in_specs=[
            pl.BlockSpec(memory_space=pltpu.SMEM),
            pl.BlockSpec(memory_space=pltpu.SMEM),
            pl.BlockSpec((4, 8, 128), lambda i: (0, i, 0)),
        ],
        out_specs=[
            pl.BlockSpec((8, 128), lambda i: (i, 0)),
            pl.BlockSpec((8, 128), lambda i: (i, 0)),
            pl.BlockSpec((8, 128), lambda i: (i, 0)),
            pl.BlockSpec((8, 128), lambda i: (i, 0)),
            pl.BlockSpec((16, 8, 128), lambda i: (0, i, 0)),
            pl.BlockSpec((16, 8, 128), lambda i: (0, i, 0)),
        ],
        out_shape=[
            jax.ShapeDtypeStruct((128, 128), jnp.float32),
            jax.ShapeDtypeStruct((128, 128), jnp.float32),
            jax.ShapeDtypeStruct((128, 128), jnp.float32),
            jax.ShapeDtypeStruct((128, 128), jnp.float32),
            jax.ShapeDtypeStruct((16, 128, 128), jnp.float32),
            jax.ShapeDtypeStruct((16, 128, 128), jnp.float32),
        ],
    )(minv, k2, g)

    p1 = q1.reshape(16, BATCH).T[:, :DUR_N]
    p2 = q2.reshape(16, BATCH).T[:, :DUR_N]

    return (
        b1.reshape(BATCH),
        d1.reshape(BATCH),
        b2.reshape(BATCH),
        d2.reshape(BATCH),
        p1,
        p2,
        anchored,
    )


# speedup vs baseline: 35.2313x; 1.0909x over previous
import functools
import math

import jax
import jax.numpy as jnp
from jax import lax
from jax.experimental import pallas as pl
from jax.experimental.pallas import tpu as pltpu
from jax.experimental.pallas import tpu_sc as plsc

NPRED = 1_000_000
BATCH = 16384
DUR_N = 11
TOTAL = float(DUR_N - 1)

_PSTR = 7936
_PROWS = _PSTR * 128
_VR = 2 * _PSTR
_PW = _PSTR * 256
_RBLK = 1984
_GRID = _VR // _RBLK

_NC, _NS = 2, 16
_NW = _NC * _NS
_BPW = BATCH // _NW

_EPS = float(jnp.finfo(jnp.float32).eps)
_LOGC = [
    math.lgamma(DUR_N) - math.lgamma(j + 1.0) - math.lgamma(TOTAL - j + 1.0)
    for j in range(DUR_N)
]


def _minanch_body(x_ref, o_ref, acc_ref):
    g = pl.program_id(0)
    x = x_ref[...]
    sq = x * x
    row = lax.broadcasted_iota(jnp.int32, x.shape, 0)

    @pl.when(g < _GRID)
    def _():
        m = jnp.min(jnp.where(row % 2 == 0, sq, jnp.inf))

        @pl.when(g == 0)
        def _():
            acc_ref[0, 0] = m

        acc_ref[0, 0] = jnp.minimum(acc_ref[0, 0], m)

    @pl.when(g >= _GRID)
    def _():
        o_ref[...] = jnp.where(row % 2 == 0, sq - acc_ref[0, 0], sq)


def _minanch_out(x_ref, o_ref, mo_ref, acc_ref):
    _minanch_body(x_ref, o_ref, acc_ref)

    @pl.when(pl.program_id(0) == _GRID - 1)
    def _():
        mo_ref[0, 0] = acc_ref[0, 0]


def _gather_sc(view1d, idx1, idx2):
    mesh = plsc.VectorSubcoreMesh(core_axis_name="c", subcore_axis_name="s")

    @functools.partial(
        pl.kernel,
        mesh=mesh,
        out_type=jax.ShapeDtypeStruct((4, 128, 128), jnp.float32),
        scratch_types=[
            pltpu.VMEM((_BPW,), jnp.int32),
            pltpu.VMEM((4, 4, 128), jnp.int32),
            pltpu.VMEM((16, 128), jnp.float32),
            pltpu.SemaphoreType.DMA,
        ],
    )
    def kg(tab, i1, i2, g_out, raw_v, adr_v, rows_v, sem):
        wid = lax.axis_index("s") * _NC + lax.axis_index("c")
        for t, src in enumerate((i1, i2)):
            pltpu.sync_copy(src.at[pl.ds(wid * _BPW, _BPW)], raw_v)
            for i in range(_BPW // 16):
                v = raw_v[pl.ds(i * 16, 16)]
                a0 = (v >> 7) * 256 + (v & 127)
                rr, cc = i // 8, (i % 8) * 16
                adr_v[2 * t, rr, pl.ds(cc, 16)] = a0
                adr_v[2 * t + 1, rr, pl.ds(cc, 16)] = a0 + 128
        gd = [
            pltpu.async_copy(tab.at[adr_v.at[r, j]], rows_v.at[4 * r + j],
                             sem)
            for r in range(4) for j in range(4)
        ]
        for d in gd:
            d.wait()
        for r in range(4):
            pltpu.sync_copy(rows_v.at[pl.ds(4 * r, 4), :],
                            g_out.at[r, pl.ds(wid * 4, 4), :])

    return kg(view1d, idx1, idx2)


def _small_body(m_ref, k_ref, g_ref,
                b1_ref, d1_ref, b2_ref, d2_ref, q1_ref, q2_ref):
    minv = m_ref[0, 0]
    kk = k_ref[0, 0]
    for t, (b_ref, d_ref, q_ref) in enumerate(
            ((b1_ref, d1_ref, q1_ref), (b2_ref, d2_ref, q2_ref))):
        a = g_ref[2 * t]
        d = g_ref[2 * t + 1]
        dur = d * d
        b_ref[...] = a * a - minv
        d_ref[...] = dur
        x = kk * jnp.log(dur)
        p = jax.nn.sigmoid(x)
        p = jnp.clip(p, _EPS, 1.0 - _EPS)
        logits = jnp.log(p) - jnp.log1p(-p)
        neg_max = jnp.minimum(logits, 0.0)
        base = TOTAL * neg_max - TOTAL * jnp.log(
            jnp.exp(neg_max) + jnp.exp(-logits + neg_max))
        for j in range(DUR_N):
            q_ref[j // 8, :, j % 8, :] = _LOGC[j] + float(j) * logits + base
        for j in range(DUR_N, 16):
            q_ref[j // 8, :, j % 8, :] = jnp.zeros_like(base)


def kernel(idx1, idx2, pred_tensor, k):
    padded = jnp.pad(pred_tensor, ((0, _PROWS - NPRED), (0, 0)),
                     constant_values=jnp.inf)
    viewp = (padded.reshape(_PSTR, 128, 2)
             .transpose(0, 2, 1)
             .reshape(_VR, 128))
    view1d = viewp.reshape(_PW)

    g = _gather_sc(view1d, idx1, idx2)

    anch_v, minv = pl.pallas_call(
        _minanch_out,
        grid=(2 * _GRID,),
        in_specs=[pl.BlockSpec((_RBLK, 128), lambda g: (g % _GRID, 0))],
        out_specs=[
            pl.BlockSpec((_RBLK, 128),
                         lambda g: ((g >= _GRID) * (g - _GRID), 0)),
            pl.BlockSpec(memory_space=pltpu.SMEM),
        ],
        out_shape=[
            jax.ShapeDtypeStruct((_VR, 128), jnp.float32),
            jax.ShapeDtypeStruct((1, 1), jnp.float32),
        ],
        scratch_shapes=[pltpu.SMEM((1, 1), jnp.float32)],
    )(viewp)

    anchored = (anch_v.reshape(_PSTR, 2, 128)
                .transpose(0, 2, 1)
                .reshape(_PROWS, 2)[:NPRED])

    k2 = k.reshape(1, 1)
    b1, d1, b2, d2, q1, q2 = pl.pallas_call(
        _small_body,
        grid=(16,),
        in_specs=[
            pl.BlockSpec(memory_space=pltpu.SMEM),
            pl.BlockSpec(memory_space=pltpu.SMEM),
            pl.BlockSpec((4, 8, 128), lambda i: (0, i, 0)),
        ],
        out_specs=[
            pl.BlockSpec((8, 128), lambda i: (i, 0)),
            pl.BlockSpec((8, 128), lambda i: (i, 0)),
            pl.BlockSpec((8, 128), lambda i: (i, 0)),
            pl.BlockSpec((8, 128), lambda i: (i, 0)),
            pl.BlockSpec((2, 8, 8, 128), lambda i: (0, i, 0, 0)),
            pl.BlockSpec((2, 8, 8, 128), lambda i: (0, i, 0, 0)),
        ],
        out_shape=[
            jax.ShapeDtypeStruct((128, 128), jnp.float32),
            jax.ShapeDtypeStruct((128, 128), jnp.float32),
            jax.ShapeDtypeStruct((128, 128), jnp.float32),
            jax.ShapeDtypeStruct((128, 128), jnp.float32),
            jax.ShapeDtypeStruct((2, 128, 8, 128), jnp.float32),
            jax.ShapeDtypeStruct((2, 128, 8, 128), jnp.float32),
        ],
    )(minv, k2, g)

    p1 = (q1.transpose(0, 2, 1, 3).reshape(16, BATCH).T)[:, :DUR_N]
    p2 = (q2.transpose(0, 2, 1, 3).reshape(16, BATCH).T)[:, :DUR_N]

    return (
        b1.reshape(BATCH),
        d1.reshape(BATCH),
        b2.reshape(BATCH),
        d2.reshape(BATCH),
        p1,
        p2,
        anchored,
    )


# speedup vs baseline: 40.8373x; 1.1591x over previous
import functools
import math

import jax
import jax.numpy as jnp
from jax import lax
from jax.experimental import pallas as pl
from jax.experimental.pallas import tpu as pltpu
from jax.experimental.pallas import tpu_sc as plsc

NPRED = 1_000_000
BATCH = 16384
DUR_N = 11
TOTAL = float(DUR_N - 1)

_PSTR = 7936
_PROWS = _PSTR * 128
_VR = 2 * _PSTR
_PW = _PSTR * 256
_RBLK = 1984
_GRID = _VR // _RBLK

_NC, _NS = 2, 16
_NW = _NC * _NS
_BPW = BATCH // _NW

_EPS = float(jnp.finfo(jnp.float32).eps)
_LOGC = [
    math.lgamma(DUR_N) - math.lgamma(j + 1.0) - math.lgamma(TOTAL - j + 1.0)
    for j in range(DUR_N)
]


def _minanch_body(x_ref, o_ref, acc_ref):
    g = pl.program_id(0)
    x = x_ref[...]
    sq = x * x
    row = lax.broadcasted_iota(jnp.int32, x.shape, 0)

    @pl.when(g < _GRID)
    def _():
        m = jnp.min(jnp.where(row % 2 == 0, sq, jnp.inf))

        @pl.when(g == 0)
        def _():
            acc_ref[0, 0] = m

        acc_ref[0, 0] = jnp.minimum(acc_ref[0, 0], m)

    @pl.when(g >= _GRID)
    def _():
        o_ref[...] = jnp.where(row % 2 == 0, sq - acc_ref[0, 0], sq)


def _minanch_out(x_ref, o_ref, mo_ref, acc_ref):
    _minanch_body(x_ref, o_ref, acc_ref)

    @pl.when(pl.program_id(0) == _GRID - 1)
    def _():
        mo_ref[0, 0] = acc_ref[0, 0]


def _gather_sc(view1d, idx1, idx2):
    mesh = plsc.VectorSubcoreMesh(core_axis_name="c", subcore_axis_name="s")

    @functools.partial(
        pl.kernel,
        mesh=mesh,
        out_type=jax.ShapeDtypeStruct((4, 128, 128), jnp.float32),
        scratch_types=[
            pltpu.VMEM((_BPW,), jnp.int32),
            pltpu.VMEM((4, 4, 128), jnp.int32),
            pltpu.VMEM((16, 128), jnp.float32),
            pltpu.SemaphoreType.DMA,
        ],
    )
    def kg(tab, i1, i2, g_out, raw_v, adr_v, rows_v, sem):
        wid = lax.axis_index("s") * _NC + lax.axis_index("c")
        for t, src in enumerate((i1, i2)):
            pltpu.sync_copy(src.at[pl.ds(wid * _BPW, _BPW)], raw_v)
            for i in range(_BPW // 16):
                v = raw_v[pl.ds(i * 16, 16)]
                a0 = (v >> 7) * 256 + (v & 127)
                rr, cc = i // 8, (i % 8) * 16
                adr_v[2 * t, rr, pl.ds(cc, 16)] = a0
                adr_v[2 * t + 1, rr, pl.ds(cc, 16)] = a0 + 128
        gd = [
            pltpu.async_copy(tab.at[adr_v.at[r, j]], rows_v.at[4 * r + j],
                             sem)
            for r in range(4) for j in range(4)
        ]
        for d in gd:
            d.wait()
        for r in range(4):
            pltpu.sync_copy(rows_v.at[pl.ds(4 * r, 4), :],
                            g_out.at[r, pl.ds(wid * 4, 4), :])

    return kg(view1d, idx1, idx2)


def _small_body(m_ref, k_ref, g_ref,
                b1_ref, d1_ref, b2_ref, d2_ref, q1_ref, q2_ref):
    minv = m_ref[0, 0]
    kk = k_ref[0, 0]
    for t, (b_ref, d_ref, q_ref) in enumerate(
            ((b1_ref, d1_ref, q1_ref), (b2_ref, d2_ref, q2_ref))):
        a = g_ref[2 * t]
        d = g_ref[2 * t + 1]
        dur = d * d
        b_ref[...] = a * a - minv
        d_ref[...] = dur
        x = kk * jnp.log(dur)
        p = jax.nn.sigmoid(x)
        p = jnp.clip(p, _EPS, 1.0 - _EPS)
        logits = jnp.log(p) - jnp.log1p(-p)
        neg_max = jnp.minimum(logits, 0.0)
        base = TOTAL * neg_max - TOTAL * jnp.log(
            jnp.exp(neg_max) + jnp.exp(-logits + neg_max))
        for j in range(DUR_N):
            q_ref[j // 8, :, j % 8, :] = _LOGC[j] + float(j) * logits + base
        for j in range(DUR_N, 16):
            q_ref[j // 8, :, j % 8, :] = jnp.zeros_like(base)


def kernel(idx1, idx2, pred_tensor, k):
    padded = jnp.pad(pred_tensor, ((0, _PROWS - NPRED), (0, 0)),
                     constant_values=jnp.inf)
    viewp = (padded.reshape(_PSTR, 128, 2)
             .transpose(0, 2, 1)
             .reshape(_VR, 128))
    view1d = viewp.reshape(_PW)

    g = _gather_sc(view1d, idx1, idx2)

    def _minanch_one(x_ref, o_ref, mo_ref):
        x = x_ref[...]
        sq = x * x
        row = lax.broadcasted_iota(jnp.int32, x.shape, 0)
        m = jnp.min(jnp.where(row % 2 == 0, sq, jnp.inf))
        mo_ref[0, 0] = m
        o_ref[...] = jnp.where(row % 2 == 0, sq - m, sq)

    anch_v, minv = pl.pallas_call(
        _minanch_one,
        in_specs=[pl.BlockSpec((_VR, 128), lambda: (0, 0))],
        out_specs=[
            pl.BlockSpec((_VR, 128), lambda: (0, 0)),
            pl.BlockSpec(memory_space=pltpu.SMEM),
        ],
        out_shape=[
            jax.ShapeDtypeStruct((_VR, 128), jnp.float32),
            jax.ShapeDtypeStruct((1, 1), jnp.float32),
        ],
    )(viewp)

    anchored = (anch_v.reshape(_PSTR, 2, 128)
                .transpose(0, 2, 1)
                .reshape(_PROWS, 2)[:NPRED])

    k2 = k.reshape(1, 1)
    b1, d1, b2, d2, q1, q2 = pl.pallas_call(
        _small_body,
        grid=(16,),
        in_specs=[
            pl.BlockSpec(memory_space=pltpu.SMEM),
            pl.BlockSpec(memory_space=pltpu.SMEM),
            pl.BlockSpec((4, 8, 128), lambda i: (0, i, 0)),
        ],
        out_specs=[
            pl.BlockSpec((8, 128), lambda i: (i, 0)),
            pl.BlockSpec((8, 128), lambda i: (i, 0)),
            pl.BlockSpec((8, 128), lambda i: (i, 0)),
            pl.BlockSpec((8, 128), lambda i: (i, 0)),
            pl.BlockSpec((2, 8, 8, 128), lambda i: (0, i, 0, 0)),
            pl.BlockSpec((2, 8, 8, 128), lambda i: (0, i, 0, 0)),
        ],
        out_shape=[
            jax.ShapeDtypeStruct((128, 128), jnp.float32),
            jax.ShapeDtypeStruct((128, 128), jnp.float32),
            jax.ShapeDtypeStruct((128, 128), jnp.float32),
            jax.ShapeDtypeStruct((128, 128), jnp.float32),
            jax.ShapeDtypeStruct((2, 128, 8, 128), jnp.float32),
            jax.ShapeDtypeStruct((2, 128, 8, 128), jnp.float32),
        ],
    )(minv, k2, g)

    p1 = (q1.transpose(0, 2, 1, 3).reshape(16, BATCH).T)[:, :DUR_N]
    p2 = (q2.transpose(0, 2, 1, 3).reshape(16, BATCH).T)[:, :DUR_N]

    return (
        b1.reshape(BATCH),
        d1.reshape(BATCH),
        b2.reshape(BATCH),
        d2.reshape(BATCH),
        p1,
        p2,
        anchored,
    )


# speedup vs baseline: 46.4168x; 1.1366x over previous
import functools
import math

import jax
import jax.numpy as jnp
from jax import lax
from jax.experimental import pallas as pl
from jax.experimental.pallas import tpu as pltpu
from jax.experimental.pallas import tpu_sc as plsc

NPRED = 1_000_000
BATCH = 16384
DUR_N = 11
TOTAL = float(DUR_N - 1)

_PSTR = 7936
_PROWS = _PSTR * 128
_VR = 2 * _PSTR
_PW = _PSTR * 256
_RBLK = 1984
_GRID = _VR // _RBLK

_NC, _NS = 2, 16
_NW = _NC * _NS
_BPW = BATCH // _NW

_EPS = float(jnp.finfo(jnp.float32).eps)
_LOGC = [
    math.lgamma(DUR_N) - math.lgamma(j + 1.0) - math.lgamma(TOTAL - j + 1.0)
    for j in range(DUR_N)
]


def _minanch_body(x_ref, o_ref, acc_ref):
    g = pl.program_id(0)
    x = x_ref[...]
    sq = x * x
    row = lax.broadcasted_iota(jnp.int32, x.shape, 0)

    @pl.when(g < _GRID)
    def _():
        m = jnp.min(jnp.where(row % 2 == 0, sq, jnp.inf))

        @pl.when(g == 0)
        def _():
            acc_ref[0, 0] = m

        acc_ref[0, 0] = jnp.minimum(acc_ref[0, 0], m)

    @pl.when(g >= _GRID)
    def _():
        o_ref[...] = jnp.where(row % 2 == 0, sq - acc_ref[0, 0], sq)


def _minanch_out(x_ref, o_ref, mo_ref, acc_ref):
    _minanch_body(x_ref, o_ref, acc_ref)

    @pl.when(pl.program_id(0) == _GRID - 1)
    def _():
        mo_ref[0, 0] = acc_ref[0, 0]


def _gather_sc(view1d, idx1, idx2):
    mesh = plsc.VectorSubcoreMesh(core_axis_name="c", subcore_axis_name="s")

    @functools.partial(
        pl.kernel,
        mesh=mesh,
        out_type=jax.ShapeDtypeStruct((4, 128, 128), jnp.float32),
        scratch_types=[
            pltpu.VMEM((_BPW,), jnp.int32),
            pltpu.VMEM((4, 4, 128), jnp.int32),
            pltpu.VMEM((16, 128), jnp.float32),
            pltpu.SemaphoreType.DMA,
        ],
    )
    def kg(tab, i1, i2, g_out, raw_v, adr_v, rows_v, sem):
        wid = lax.axis_index("s") * _NC + lax.axis_index("c")
        for t, src in enumerate((i1, i2)):
            pltpu.sync_copy(src.at[pl.ds(wid * _BPW, _BPW)], raw_v)
            for i in range(_BPW // 16):
                v = raw_v[pl.ds(i * 16, 16)]
                a0 = (v >> 7) * 256 + (v & 127)
                rr, cc = i // 8, (i % 8) * 16
                adr_v[2 * t, rr, pl.ds(cc, 16)] = a0
                adr_v[2 * t + 1, rr, pl.ds(cc, 16)] = a0 + 128
        gd = [
            pltpu.async_copy(tab.at[adr_v.at[r, j]], rows_v.at[4 * r + j],
                             sem)
            for r in range(4) for j in range(4)
        ]
        for d in gd:
            d.wait()
        for r in range(4):
            pltpu.sync_copy(rows_v.at[pl.ds(4 * r, 4), :],
                            g_out.at[r, pl.ds(wid * 4, 4), :])

    return kg(view1d, idx1, idx2)


def _small_body(m_ref, k_ref, g_ref,
                b1_ref, d1_ref, b2_ref, d2_ref, q1_ref, q2_ref):
    minv = m_ref[0, 0]
    kk = k_ref[0, 0]
    for t, (b_ref, d_ref, q_ref) in enumerate(
            ((b1_ref, d1_ref, q1_ref), (b2_ref, d2_ref, q2_ref))):
        a = g_ref[2 * t]
        d = g_ref[2 * t + 1]
        dur = d * d
        b_ref[...] = a * a - minv
        d_ref[...] = dur
        x = kk * jnp.log(dur)
        p = jax.nn.sigmoid(x)
        p = jnp.clip(p, _EPS, 1.0 - _EPS)
        logits = jnp.log(p) - jnp.log1p(-p)
        neg_max = jnp.minimum(logits, 0.0)
        base = TOTAL * neg_max - TOTAL * jnp.log(
            jnp.exp(neg_max) + jnp.exp(-logits + neg_max))
        for j in range(DUR_N):
            q_ref[j // 8, :, j % 8, :] = _LOGC[j] + float(j) * logits + base
        for j in range(DUR_N, 16):
            q_ref[j // 8, :, j % 8, :] = jnp.zeros_like(base)


def kernel(idx1, idx2, pred_tensor, k):
    padded = jnp.pad(pred_tensor, ((0, _PROWS - NPRED), (0, 0)),
                     constant_values=jnp.inf)
    viewp = (padded.reshape(_PSTR, 128, 2)
             .transpose(0, 2, 1)
             .reshape(_VR, 128))
    view1d = viewp.reshape(_PW)

    g = _gather_sc(view1d, idx1, idx2)

    def _minanch_one(x_ref, o_ref, mo_ref):
        x = x_ref[...]
        sq = x * x
        row = lax.broadcasted_iota(jnp.int32, x.shape, 0)
        m = jnp.min(jnp.where(row % 2 == 0, sq, jnp.inf))
        mo_ref[0, 0] = m
        o_ref[...] = jnp.where(row % 2 == 0, sq - m, sq)

    anch_v, minv = pl.pallas_call(
        _minanch_one,
        in_specs=[pl.BlockSpec((_VR, 128), lambda: (0, 0))],
        out_specs=[
            pl.BlockSpec((_VR, 128), lambda: (0, 0)),
            pl.BlockSpec(memory_space=pltpu.SMEM),
        ],
        out_shape=[
            jax.ShapeDtypeStruct((_VR, 128), jnp.float32),
            jax.ShapeDtypeStruct((1, 1), jnp.float32),
        ],
    )(viewp)

    anchored = (anch_v.reshape(_PSTR, 2, 128)
                .transpose(0, 2, 1)
                .reshape(_PROWS, 2)[:NPRED])

    k2 = k.reshape(1, 1)
    b1, d1, b2, d2, q1, q2 = pl.pallas_call(
        _small_body,
        in_specs=[
            pl.BlockSpec(memory_space=pltpu.SMEM),
            pl.BlockSpec(memory_space=pltpu.SMEM),
            pl.BlockSpec((4, 128, 128), lambda: (0, 0, 0)),
        ],
        out_specs=[
            pl.BlockSpec((128, 128), lambda: (0, 0)),
            pl.BlockSpec((128, 128), lambda: (0, 0)),
            pl.BlockSpec((128, 128), lambda: (0, 0)),
            pl.BlockSpec((128, 128), lambda: (0, 0)),
            pl.BlockSpec((2, 128, 8, 128), lambda: (0, 0, 0, 0)),
            pl.BlockSpec((2, 128, 8, 128), lambda: (0, 0, 0, 0)),
        ],
        out_shape=[
            jax.ShapeDtypeStruct((128, 128), jnp.float32),
            jax.ShapeDtypeStruct((128, 128), jnp.float32),
            jax.ShapeDtypeStruct((128, 128), jnp.float32),
            jax.ShapeDtypeStruct((128, 128), jnp.float32),
            jax.ShapeDtypeStruct((2, 128, 8, 128), jnp.float32),
            jax.ShapeDtypeStruct((2, 128, 8, 128), jnp.float32),
        ],
    )(minv, k2, g)

    p1 = (q1.transpose(0, 2, 1, 3).reshape(16, BATCH).T)[:, :DUR_N]
    p2 = (q2.transpose(0, 2, 1, 3).reshape(16, BATCH).T)[:, :DUR_N]

    return (
        b1.reshape(BATCH),
        d1.reshape(BATCH),
        b2.reshape(BATCH),
        d2.reshape(BATCH),
        p1,
        p2,
        anchored,
    )
